# speedup vs baseline: 1.9073x; 1.9073x over previous
import jax
import jax.numpy as jnp
from jax import lax
from jax.experimental import pallas as pl
from jax.experimental.pallas import tpu as pltpu

B = 64
C = 64
HW = 1024
K = 512
N = B * HW
NELEM = N * C


def _tc_body(z_ref, emb_ref, zq_ref, idx_ref, loss_ref, perp_ref,
             acc, mean_s, std_s, embt_s, err_s, cnt):
    i = pl.program_id(0)

    @pl.when(i == 0)
    def _init():
        acc[...] = jnp.zeros_like(acc)

    @pl.when(i < B)
    def _phase_sum():
        acc[...] += z_ref[0]

    @pl.when(i == B)
    def _fin_mean():
        mean_s[...] = jnp.sum(acc[...], axis=1, keepdims=True) / N
        acc[...] = jnp.zeros_like(acc)

    @pl.when(jnp.logical_and(i >= B, i < 2 * B))
    def _phase_sq():
        d = z_ref[0] - mean_s[...]
        acc[...] += d * d

    @pl.when(i == 2 * B)
    def _fin_std():
        var = jnp.sum(acc[...], axis=1, keepdims=True) / (N - 1)
        std_s[...] = jnp.maximum(jnp.sqrt(var), 1e-5)
        embt_s[...] = emb_ref[...].T
        err_s[0, 0] = 0.0
        cnt[...] = jnp.zeros_like(cnt)

    @pl.when(i >= 2 * B)
    def _phase_main():
        zn = z_ref[0] / std_s[...]
        emb = emb_ref[...]
        mm = lax.dot_general(emb, zn, (((1,), (0,)), ((), ())),
                             preferred_element_type=jnp.float32)
        esq = jnp.sum(emb * emb, axis=1, keepdims=True)
        zsq = jnp.sum(zn * zn, axis=0, keepdims=True)
        dist = (esq - 2.0 * mm) + zsq
        md = jnp.min(dist, axis=0)
        kio = lax.broadcasted_iota(jnp.int32, (K, HW), 0)
        idx = jnp.min(jnp.where(dist == md[None, :], kio, K), axis=0)
        err_s[0, 0] += jnp.sum(md)
        oh = (kio == idx[None, :]).astype(jnp.float32)
        cnt[...] += jnp.sum(oh, axis=1, keepdims=True)
        zq_ref[0] = lax.dot_general(embt_s[...], oh, (((1,), (0,)), ((), ())),
                                    preferred_element_type=jnp.float32)
        idx_ref[0, 0, :] = idx

    @pl.when(i == 3 * B - 1)
    def _finalize():
        loss_ref[0, 0] = 1.25 * err_s[0, 0] / NELEM
        p = cnt[...] / N
        plogp = p * jnp.log(jnp.maximum(p, 1e-10))
        perp_ref[0, 0] = jnp.exp(-jnp.sum(plogp))


def kernel(z_e, emb_w):
    z3 = z_e.reshape(B, C, HW)
    zq, idx3, loss, perp = pl.pallas_call(
        _tc_body,
        grid=(3 * B,),
        in_specs=[
            pl.BlockSpec((1, C, HW), lambda i: (i % B, 0, 0)),
            pl.BlockSpec((K, C), lambda i: (0, 0)),
        ],
        out_specs=[
            pl.BlockSpec((1, C, HW), lambda i: (jnp.maximum(i - 2 * B, 0), 0, 0)),
            pl.BlockSpec((1, 1, HW), lambda i: (jnp.maximum(i - 2 * B, 0), 0, 0)),
            pl.BlockSpec(memory_space=pltpu.SMEM),
            pl.BlockSpec(memory_space=pltpu.SMEM),
        ],
        out_shape=[
            jax.ShapeDtypeStruct((B, C, HW), jnp.float32),
            jax.ShapeDtypeStruct((B, 1, HW), jnp.int32),
            jax.ShapeDtypeStruct((1, 1), jnp.float32),
            jax.ShapeDtypeStruct((1, 1), jnp.float32),
        ],
        scratch_shapes=[
            pltpu.VMEM((C, HW), jnp.float32),
            pltpu.VMEM((C, 1), jnp.float32),
            pltpu.VMEM((C, 1), jnp.float32),
            pltpu.VMEM((C, K), jnp.float32),
            pltpu.SMEM((1, 1), jnp.float32),
            pltpu.VMEM((K, 1), jnp.float32),
        ],
    )(z3, emb_w)
    z_q_st = zq.reshape(z_e.shape)
    indices = idx3.reshape(B, 32, 32)
    return (z_q_st, loss[0, 0], perp[0, 0], indices)
